# baseline (speedup 1.0000x reference)
import functools
import jax
import jax.numpy as jnp
from jax import lax
from jax.experimental import pallas as pl
from jax.experimental.pallas import tpu as pltpu, tpu_sc as plsc

GRID = 0.0078125
SCALE_MULT = 0.05
N = 8192
P = 1024
C = 18

NC = 2
NS = 16
NW = NC * NS
NCELL = 512
CPW = NCELL // NW
CAP = 64
LSTR = 80
PCAP = 448
PQ = 4
PQB = PCAP // PQ

_mesh = plsc.VectorSubcoreMesh(core_axis_name="c", subcore_axis_name="s")


def _sload(ref, i):
    return ref[pl.ds(i, 16)][0]


def _sstore(ref, i, val, dtype):
    lane0 = lax.iota(jnp.int32, 16) == 0
    plsc.store_scatter(ref, [jnp.full((16,), i, jnp.int32)],
                       jnp.full((16,), val, dtype), mask=lane0)


@functools.partial(
    pl.kernel,
    out_type=jax.ShapeDtypeStruct((N * 32,), jnp.float32),
    mesh=_mesh,
    compiler_params=pltpu.CompilerParams(needs_layout_passes=False),
    scratch_types=[
        pltpu.VMEM((10 * P,), jnp.float32),
        pltpu.VMEM((10 * P,), jnp.int32),
        pltpu.VMEM((P * 32,), jnp.float32),
        pltpu.VMEM((24 * P,), jnp.int32),
        pltpu.VMEM((N,), jnp.int32),
        pltpu.VMEM((3 * N + 16,), jnp.float32),
        pltpu.VMEM((CPW * LSTR,), jnp.int32),
        pltpu.VMEM((CPW + 16,), jnp.int32),
        pltpu.VMEM((PCAP + 32,), jnp.int32),
        pltpu.VMEM((PCAP + 32,), jnp.int32),
        pltpu.VMEM((PCAP * 32 + 32,), jnp.float32),
        pltpu.VMEM((96,), jnp.float32),
        pltpu.VMEM((96,), jnp.int32),
        pltpu.SemaphoreType.DMA,
    ],
)
def _sc_agg(gf_hbm, gi_hbm, sem_hbm, ptf_hbm, pti_hbm, out_hbm,
            gf_v, gi_v, sem_v, xm_v, cid_v, pw_v, ll_v, lc_v,
            pidf_v, pcidf_v, stag_v, hw_v, hg_v, dsem):
    cidx = lax.axis_index("c")
    sidx = lax.axis_index("s")
    wid = sidx * NC + cidx

    pltpu.sync_copy(gf_hbm, gf_v)
    pltpu.sync_copy(gi_hbm, gi_v)
    pltpu.sync_copy(sem_hbm, sem_v)
    pltpu.sync_copy(pti_hbm.at[pl.ds(3 * N, N)], cid_v)
    pltpu.sync_copy(ptf_hbm, pw_v.at[pl.ds(0, 3 * N)])

    def _axis_body(aq, _):
        a = aq // 8
        q = aq - a * 8

        def _j(j, __):
            clo = gi_v[pl.ds(2 * a * P + j * 16, 16)]
            chi = gi_v[pl.ds((2 * a + 1) * P + j * 16, 16)]
            m = ((clo <= q) & (q <= chi)).astype(jnp.int32)
            xm_v[pl.ds(aq * P + j * 16, 16)] = m
            return __
        return lax.fori_loop(0, P // 16, _j, _)
    lax.fori_loop(0, 24, _axis_body, 0)

    def _cell_body(k, _):
        cell = wid * CPW + k
        cx = cell // 64
        cy = (cell // 8) - cx * 8
        cz = cell - cx * 64 - cy * 8

        def _j(j, off):
            xv = xm_v[pl.ds((0 * 8 + cx) * P + j * 16, 16)]
            yv = xm_v[pl.ds((1 * 8 + cy) * P + j * 16, 16)]
            zv = xm_v[pl.ds((2 * 8 + cz) * P + j * 16, 16)]
            m = (xv & yv & zv) == 1
            ids = lax.iota(jnp.int32, 16) + j * 16
            mi = m.astype(jnp.int32)
            pos = plsc.cumsum(mi) - mi + (k * LSTR + off)
            plsc.store_scatter(ll_v, [pos], ids, mask=m)
            off = off + jnp.sum(mi)
            return jnp.minimum(off, CAP)
        cnt = lax.fori_loop(0, P // 16, _j, jnp.int32(0))
        _sstore(lc_v, k, cnt, jnp.int32)
        return _
    lax.fori_loop(0, CPW, _cell_body, 0)

    pltpu.sync_copy(pti_hbm.at[pl.ds(0, 3 * N)], xm_v)

    def _pt_scan(v, np_):
        cv = cid_v[pl.ds(v * 16, 16)]
        m = (cv >> 4) == wid
        ids = lax.iota(jnp.int32, 16) + v * 16
        mi = m.astype(jnp.int32)
        pos = plsc.cumsum(mi) - mi + np_
        plsc.store_scatter(pidf_v, [pos], ids, mask=m)
        plsc.store_scatter(pcidf_v, [pos], cv, mask=m)
        np_ = np_ + jnp.sum(mi)
        return jnp.minimum(np_, PCAP)
    npts = lax.fori_loop(0, N // 16, _pt_scan, jnp.int32(0))

    def _pt_body(i, _):
        pid = _sload(pidf_v, i)
        pcid = _sload(pcidf_v, i)
        k = pcid - wid * CPW
        cnt = _sload(lc_v, k)
        px = _sload(pw_v, pid)
        py = _sload(pw_v, N + pid)
        pz = _sload(pw_v, 2 * N + pid)
        pix = _sload(xm_v, pid)
        piy = _sload(xm_v, N + pid)
        piz = _sload(xm_v, 2 * N + pid)
        nj = (cnt + 15) // 16

        def _j(j, hoff):
            idx = ll_v[pl.ds(k * LSTR + j * 16, 16)]
            valid = (lax.iota(jnp.int32, 16) + j * 16) < cnt
            gidx = jnp.where(valid, idx, 0)
            mix = plsc.load_gather(gi_v, [gidx + 6 * P])
            miy = plsc.load_gather(gi_v, [gidx + 7 * P])
            miz = plsc.load_gather(gi_v, [gidx + 8 * P])
            rr = plsc.load_gather(gi_v, [gidx + 9 * P])
            within = ((jnp.abs(pix - mix) <= rr)
                      & (jnp.abs(piy - miy) <= rr)
                      & (jnp.abs(piz - miz) <= rr))
            mask = valid & within
            mux = plsc.load_gather(gf_v, [gidx])
            muy = plsc.load_gather(gf_v, [gidx + P])
            muz = plsc.load_gather(gf_v, [gidx + 2 * P])
            c0 = plsc.load_gather(gf_v, [gidx + 3 * P])
            c1 = plsc.load_gather(gf_v, [gidx + 4 * P])
            c2 = plsc.load_gather(gf_v, [gidx + 5 * P])
            c3 = plsc.load_gather(gf_v, [gidx + 6 * P])
            c4 = plsc.load_gather(gf_v, [gidx + 7 * P])
            c5 = plsc.load_gather(gf_v, [gidx + 8 * P])
            opg = plsc.load_gather(gf_v, [gidx + 9 * P])
            dx = px - mux
            dy = py - muy
            dz = pz - muz
            power = (-0.5 * (c0 * dx * dx + c1 * dy * dy + c2 * dz * dz)
                     - c3 * dx * dy - c4 * dy * dz - c5 * dx * dz)
            w = opg * jnp.exp(power)
            w = jnp.where(mask, w, 0.0)
            mi = mask.astype(jnp.int32)
            pos = plsc.cumsum(mi) - mi + hoff
            plsc.store_scatter(hw_v, [pos], w, mask=mask)
            plsc.store_scatter(hg_v, [pos], gidx, mask=mask)
            return hoff + jnp.sum(mi)
        hcnt = lax.fori_loop(0, nj, _j, jnp.int32(0))

        def _hit(h, acc):
            a0, a1 = acc
            wv = _sload(hw_v, h)
            g = _sload(hg_v, h)
            a0 = a0 + wv * sem_v[pl.ds(g * 32, 16)]
            a1 = a1 + wv * sem_v[pl.ds(g * 32 + 16, 16)]
            return (a0, a1)
        acc0, acc1 = lax.fori_loop(
            0, hcnt, _hit,
            (jnp.zeros((16,), jnp.float32), jnp.zeros((16,), jnp.float32)))

        stag_v[pl.ds(i * 32, 16)] = acc0
        stag_v[pl.ds(i * 32 + 16, 16)] = acc1
        return _
    lax.fori_loop(0, npts, _pt_body, 0)

    def _out(i, _):
        pid = _sload(pidf_v, i)
        pltpu.async_copy(stag_v.at[pl.ds(i * 32, 32)],
                         out_hbm.at[pl.ds(pid * 32, 32)], dsem)
        return _
    lax.fori_loop(0, npts, _out, 0)

    def _drain(i, _):
        pltpu.make_async_copy(stag_v.at[pl.ds(0, 32)],
                              out_hbm.at[pl.ds(0, 32)], dsem).wait()
        return _
    lax.fori_loop(0, npts, _drain, 0)


def kernel(pts, means3D, opacities, semantics, scales, cov3D, metas, origin_use):
    p = pts[0]
    mu = means3D[0]
    op = opacities[0]
    sem = semantics[0]
    sc = scales[0]
    cov = cov3D[0]

    inv_g = 1.0 / GRID
    pint = jnp.floor((p - origin_use) * inv_g).astype(jnp.int32)
    mint = jnp.floor((mu - origin_use) * inv_g).astype(jnp.int32)
    radii = jnp.ceil(jnp.max(sc, axis=-1) * (SCALE_MULT * inv_g)).astype(jnp.int32)

    pcell = ((pint[:, 0] >> 4) * 64 + (pint[:, 1] >> 4) * 8 + (pint[:, 2] >> 4))
    clo = jnp.clip((mint - radii[:, None]) >> 4, 0, 7)
    chi = jnp.clip((mint + radii[:, None]) >> 4, 0, 7)

    cov6 = cov.reshape(P, 9)[:, jnp.array([0, 4, 8, 1, 5, 2])]
    gf = jnp.concatenate([mu.T, cov6.T, op[None, :]], axis=0).reshape(-1)
    gi = jnp.stack([clo[:, 0], chi[:, 0], clo[:, 1], chi[:, 1],
                    clo[:, 2], chi[:, 2],
                    mint[:, 0], mint[:, 1], mint[:, 2], radii]).reshape(-1)
    semp = jnp.pad(sem, ((0, 0), (0, 32 - C))).reshape(-1)
    ptf = p.T.reshape(-1)
    pti = jnp.concatenate([pint.T, pcell[None, :]], axis=0).reshape(-1)

    out = _sc_agg(gf, gi, semp, ptf, pti)
    return out.reshape(N, 32)[:, :C]


# speedup vs baseline: 1.2227x; 1.2227x over previous
import functools
import jax
import jax.numpy as jnp
from jax import lax
from jax.experimental import pallas as pl
from jax.experimental.pallas import tpu as pltpu, tpu_sc as plsc

GRID = 0.0078125
SCALE_MULT = 0.05
N = 8192
P = 1024
C = 18

NC = 2
NS = 16
NW = NC * NS
NCELL = 512
CPW = NCELL // NW
CAP = 64
LSTR = 80
PCAP = 384
HITCAP = 320

_mesh = plsc.VectorSubcoreMesh(core_axis_name="c", subcore_axis_name="s")


def _sload(ref, i):
    return ref[pl.ds(i, 16)][0]


def _sstore(ref, i, val, dtype):
    lane0 = lax.iota(jnp.int32, 16) == 0
    plsc.store_scatter(ref, [jnp.full((16,), i, jnp.int32)],
                       jnp.full((16,), val, dtype), mask=lane0)


@functools.partial(
    pl.kernel,
    out_type=jax.ShapeDtypeStruct((N * 32,), jnp.float32),
    mesh=_mesh,
    compiler_params=pltpu.CompilerParams(needs_layout_passes=False),
    scratch_types=[
        pltpu.VMEM((10 * P,), jnp.float32),
        pltpu.VMEM((4 * P + 16,), jnp.int32),
        pltpu.VMEM((P * 16,), jnp.float32),
        pltpu.VMEM((P * 8 + 16,), jnp.float32),
        pltpu.VMEM((2 * P,), jnp.int32),
        pltpu.VMEM((8 * P,), jnp.int32),
        pltpu.VMEM((P + 16,), jnp.int32),
        pltpu.VMEM((24 * P,), jnp.int32),
        pltpu.VMEM((2048,), jnp.int32),
        pltpu.VMEM((3 * N + 16,), jnp.float32),
        pltpu.VMEM((CPW * LSTR,), jnp.int32),
        pltpu.VMEM((CPW + 16,), jnp.int32),
        pltpu.VMEM((PCAP + 32,), jnp.int32),
        pltpu.VMEM((PCAP + 32,), jnp.int32),
        pltpu.VMEM((PCAP * 32 + 32,), jnp.float32),
        pltpu.VMEM((HITCAP + 32,), jnp.float32),
        pltpu.VMEM((HITCAP + 32,), jnp.int32),
        pltpu.VMEM((HITCAP + 32,), jnp.int32),
        pltpu.SemaphoreType.DMA,
    ],
)
def _sc_agg(gf_hbm, gi_hbm, semA_hbm, semB_hbm, ptf_hbm, pti_hbm, out_hbm,
            gf_v, gi_v, semA_v, semB_v, ym_v, zm_v, xl_v, pi_v, cidc_v, pw_v,
            ll_v, lc_v, pidf_v, pcidf_v, stag_v, hw_v, hg_v, hil_v, dsem):
    cidx = lax.axis_index("c")
    sidx = lax.axis_index("s")
    wid = sidx * NC + cidx
    mycx = wid // 4
    cy0 = (wid * 2) % 8

    pltpu.sync_copy(gf_hbm, gf_v)
    pltpu.sync_copy(gi_hbm, gi_v.at[pl.ds(0, 4 * P)])
    pltpu.sync_copy(semA_hbm, semA_v)
    pltpu.sync_copy(semB_hbm, semB_v.at[pl.ds(0, P * 8)])
    pltpu.sync_copy(pti_hbm.at[pl.ds(0, 3 * N)], pi_v)
    pltpu.sync_copy(ptf_hbm, pw_v.at[pl.ds(0, 3 * N)])

    def _xscan(j, off):
        mix = gi_v[pl.ds(j * 16, 16)]
        rr = gi_v[pl.ds(3 * P + j * 16, 16)]
        m = ((lax.shift_right_arithmetic(mix - rr, 4) <= mycx)
             & (mycx <= lax.shift_right_arithmetic(mix + rr, 4)))
        ids = lax.iota(jnp.int32, 16) + j * 16
        mi = m.astype(jnp.int32)
        pos = plsc.cumsum(mi) - mi + off
        plsc.store_scatter(xl_v, [pos], ids, mask=m)
        return jnp.minimum(off + jnp.sum(mi), P)
    xcnt = lax.fori_loop(0, P // 16, _xscan, jnp.int32(0))

    def _yz(j, _):
        miy = gi_v[pl.ds(P + j * 16, 16)]
        miz = gi_v[pl.ds(2 * P + j * 16, 16)]
        rr = gi_v[pl.ds(3 * P + j * 16, 16)]
        ylo = lax.shift_right_arithmetic(miy - rr, 4)
        yhi = lax.shift_right_arithmetic(miy + rr, 4)
        ym_v[pl.ds(j * 16, 16)] = ((ylo <= cy0) & (cy0 <= yhi)).astype(jnp.int32)
        ym_v[pl.ds(P + j * 16, 16)] = (
            (ylo <= cy0 + 1) & (cy0 + 1 <= yhi)).astype(jnp.int32)
        zlo = lax.shift_right_arithmetic(miz - rr, 4)
        zhi = lax.shift_right_arithmetic(miz + rr, 4)
        for q in range(8):
            zm_v[pl.ds(q * P + j * 16, 16)] = (
                (zlo <= q) & (q <= zhi)).astype(jnp.int32)
        return _
    lax.fori_loop(0, P // 16, _yz, 0)

    nxv = (xcnt + 15) // 16

    def _cell_body(k, _):
        yrow = (k // 8) * P
        zrow = (k - (k // 8) * 8) * P

        def _j(j, off):
            gid = xl_v[pl.ds(j * 16, 16)]
            lane_ok = (lax.iota(jnp.int32, 16) + j * 16) < xcnt
            gid = jnp.where(lane_ok, gid, 0)
            ym = plsc.load_gather(ym_v, [gid + yrow])
            zm = plsc.load_gather(zm_v, [gid + zrow])
            m = lane_ok & ((ym & zm) == 1)
            mi = m.astype(jnp.int32)
            pos = plsc.cumsum(mi) - mi + (k * LSTR + off)
            plsc.store_scatter(ll_v, [pos], gid, mask=m)
            return jnp.minimum(off + jnp.sum(mi), CAP)
        cnt = lax.fori_loop(0, nxv, _j, jnp.int32(0))
        _sstore(lc_v, k, cnt, jnp.int32)
        return _
    lax.fori_loop(0, CPW, _cell_body, 0)

    npts = jnp.int32(0)
    for b in range(N // 2048):
        pltpu.sync_copy(pti_hbm.at[pl.ds(3 * N + b * 2048, 2048)], cidc_v)

        def _pt_scan(v, np_, _b=b):
            cv = cidc_v[pl.ds(v * 16, 16)]
            m = (cv >> 4) == wid
            ids = lax.iota(jnp.int32, 16) + (_b * 2048 + v * 16)
            mi = m.astype(jnp.int32)
            pos = plsc.cumsum(mi) - mi + np_
            plsc.store_scatter(pidf_v, [pos], ids, mask=m)
            plsc.store_scatter(pcidf_v, [pos], cv, mask=m)
            return jnp.minimum(np_ + jnp.sum(mi), PCAP)
        npts = lax.fori_loop(0, 2048 // 16, _pt_scan, npts)

    def _pt_body(i, hoff):
        pid = _sload(pidf_v, i)
        pcid = _sload(pcidf_v, i)
        k = pcid - wid * CPW
        cnt = _sload(lc_v, k)
        pix = _sload(pi_v, pid)
        piy = _sload(pi_v, N + pid)
        piz = _sload(pi_v, 2 * N + pid)
        stag_v[pl.ds(i * 32, 16)] = jnp.zeros((16,), jnp.float32)
        stag_v[pl.ds(i * 32 + 16, 16)] = jnp.zeros((16,), jnp.float32)
        nj = (cnt + 15) // 16

        def _j(j, hoff):
            idx = ll_v[pl.ds(k * LSTR + j * 16, 16)]
            valid = (lax.iota(jnp.int32, 16) + j * 16) < cnt
            gidx = jnp.where(valid, idx, 0)
            mix = plsc.load_gather(gi_v, [gidx])
            miy = plsc.load_gather(gi_v, [gidx + P])
            miz = plsc.load_gather(gi_v, [gidx + 2 * P])
            rr = plsc.load_gather(gi_v, [gidx + 3 * P])
            mask = (valid
                    & (jnp.abs(pix - mix) <= rr)
                    & (jnp.abs(piy - miy) <= rr)
                    & (jnp.abs(piz - miz) <= rr))
            mi = mask.astype(jnp.int32)
            pos = plsc.cumsum(mi) - mi + hoff
            plsc.store_scatter(hg_v, [pos], gidx, mask=mask)
            plsc.store_scatter(hil_v, [pos],
                               jnp.full((16,), i, jnp.int32), mask=mask)
            return jnp.minimum(hoff + jnp.sum(mi), HITCAP)
        return lax.fori_loop(0, nj, _j, hoff)
    nhits = lax.fori_loop(0, npts, _pt_body, jnp.int32(0))

    nhv = (nhits + 15) // 16

    def _ev(hv, _):
        lane_ok = (lax.iota(jnp.int32, 16) + hv * 16) < nhits
        gidx = jnp.where(lane_ok, hg_v[pl.ds(hv * 16, 16)], 0)
        il = jnp.where(lane_ok, hil_v[pl.ds(hv * 16, 16)], 0)
        pidv = plsc.load_gather(pidf_v, [il])
        px = plsc.load_gather(pw_v, [pidv])
        py = plsc.load_gather(pw_v, [pidv + N])
        pz = plsc.load_gather(pw_v, [pidv + 2 * N])
        mux = plsc.load_gather(gf_v, [gidx])
        muy = plsc.load_gather(gf_v, [gidx + P])
        muz = plsc.load_gather(gf_v, [gidx + 2 * P])
        c0 = plsc.load_gather(gf_v, [gidx + 3 * P])
        c1 = plsc.load_gather(gf_v, [gidx + 4 * P])
        c2 = plsc.load_gather(gf_v, [gidx + 5 * P])
        c3 = plsc.load_gather(gf_v, [gidx + 6 * P])
        c4 = plsc.load_gather(gf_v, [gidx + 7 * P])
        c5 = plsc.load_gather(gf_v, [gidx + 8 * P])
        opg = plsc.load_gather(gf_v, [gidx + 9 * P])
        dx = px - mux
        dy = py - muy
        dz = pz - muz
        power = (-0.5 * (c0 * dx * dx + c1 * dy * dy + c2 * dz * dz)
                 - c3 * dx * dy - c4 * dy * dz - c5 * dx * dz)
        hw_v[pl.ds(hv * 16, 16)] = opg * jnp.exp(power)
        return _
    lax.fori_loop(0, nhv, _ev, 0)

    lo8 = lax.iota(jnp.int32, 16) < 8

    def _hit(h, _):
        wv = _sload(hw_v, h)
        g = _sload(hg_v, h)
        il = _sload(hil_v, h)
        a0 = stag_v[pl.ds(il * 32, 16)]
        stag_v[pl.ds(il * 32, 16)] = a0 + wv * semA_v[pl.ds(g * 16, 16)]
        a1 = stag_v[pl.ds(il * 32 + 16, 16)]
        sb = jnp.where(lo8, semB_v[pl.ds(g * 8, 16)], 0.0)
        stag_v[pl.ds(il * 32 + 16, 16)] = a1 + wv * sb
        return _
    lax.fori_loop(0, nhits, _hit, 0)

    def _out(i, _):
        pid = _sload(pidf_v, i)
        pltpu.async_copy(stag_v.at[pl.ds(i * 32, 32)],
                         out_hbm.at[pl.ds(pid * 32, 32)], dsem)
        return _
    lax.fori_loop(0, npts, _out, 0)

    def _drain(i, _):
        pltpu.make_async_copy(stag_v.at[pl.ds(0, 32)],
                              out_hbm.at[pl.ds(0, 32)], dsem).wait()
        return _
    lax.fori_loop(0, npts, _drain, 0)


def kernel(pts, means3D, opacities, semantics, scales, cov3D, metas, origin_use):
    p = pts[0]
    mu = means3D[0]
    op = opacities[0]
    sem = semantics[0]
    sc = scales[0]
    cov = cov3D[0]

    inv_g = 1.0 / GRID
    pint = jnp.floor((p - origin_use) * inv_g).astype(jnp.int32)
    mint = jnp.floor((mu - origin_use) * inv_g).astype(jnp.int32)
    radii = jnp.ceil(jnp.max(sc, axis=-1) * (SCALE_MULT * inv_g)).astype(jnp.int32)

    pcell = ((pint[:, 0] >> 4) * 64 + (pint[:, 1] >> 4) * 8 + (pint[:, 2] >> 4))

    cov6 = cov.reshape(P, 9)[:, jnp.array([0, 4, 8, 1, 5, 2])]
    gf = jnp.concatenate([mu.T, cov6.T, op[None, :]], axis=0).reshape(-1)
    gi = jnp.concatenate([mint.T, radii[None, :]], axis=0).reshape(-1)
    semA = sem[:, :16].reshape(-1)
    semB = jnp.pad(sem[:, 16:], ((0, 0), (0, 6))).reshape(-1)
    ptf = p.T.reshape(-1)
    pti = jnp.concatenate([pint.T, pcell[None, :]], axis=0).reshape(-1)

    out = _sc_agg(gf, gi, semA, semB, ptf, pti)
    return out.reshape(N, 32)[:, :C]
